# baseline (speedup 1.0000x reference)
import functools

import numpy as np
import jax
import jax.numpy as jnp
from jax import lax
from jax.experimental import pallas as pl
from jax.experimental.pallas import tpu as pltpu, tpu_sc as plsc

_D = 768
_BATCH = 4
_SEQ = 2048
_ROWS = _BATCH * _SEQ

_NW = 32
_CB = 64
_ROWS_PER_W = _ROWS // _NW
_NCHUNK = _ROWS_PER_W // _CB
_VPR = _D // 16


def _sinusoidal_pe(max_len, d_model):
    pos = np.arange(max_len, dtype=np.float32)[:, None]
    div = np.exp(
        np.arange(0, d_model, 2, dtype=np.float32) * (-np.log(10000.0) / d_model)
    )
    pe = np.zeros((max_len, d_model), dtype=np.float32)
    pe[:, 0::2] = np.sin(pos * div)
    pe[:, 1::2] = np.cos(pos * div)
    return jnp.asarray(pe)


_PE = _sinusoidal_pe(_SEQ, _D)

_mesh = plsc.VectorSubcoreMesh(core_axis_name="c", subcore_axis_name="s")


@functools.partial(
    pl.kernel,
    out_type=jax.ShapeDtypeStruct((_ROWS, _D), jnp.float32),
    mesh=_mesh,
    scratch_types=[
        pltpu.VMEM((_NCHUNK, _CB), jnp.int32),
        pltpu.VMEM((_CB, _D), jnp.float32),
        pltpu.VMEM((_CB, _D), jnp.float32),
        pltpu.SemaphoreType.DMA,
    ],
)
def _emb_kernel(idx_hbm, table_hbm, pe_hbm, out_hbm, idx_v, buf, rows, sem):
    wid = lax.axis_index("s") * 2 + lax.axis_index("c")
    base = wid * _ROWS_PER_W
    pbase = base % _SEQ
    pltpu.sync_copy(idx_hbm.at[pl.ds(wid * _NCHUNK, _NCHUNK)], idx_v)
    for c in range(_NCHUNK):
        gather = pltpu.async_copy(table_hbm.at[idx_v.at[c]], rows, sem)
        pltpu.sync_copy(pe_hbm.at[pl.ds(pbase + c * _CB, _CB)], buf)
        gather.wait()

        def add_row(r):
            for j in range(_VPR):
                sl = pl.ds(j * 16, 16)
                buf[r, sl] = buf[r, sl] + rows[r, sl]

        lax.fori_loop(0, _CB, lambda r, _: (add_row(r), 0)[1], 0)
        pltpu.sync_copy(buf, out_hbm.at[pl.ds(base + c * _CB, _CB)])


def kernel(x, table):
    idx = x.reshape(_NW * _NCHUNK, _CB).astype(jnp.int32)
    out = _emb_kernel(idx, table, _PE)
    return out.reshape(_BATCH, _SEQ, _D)


# speedup vs baseline: 1.1045x; 1.1045x over previous
import functools

import numpy as np
import jax
import jax.numpy as jnp
from jax import lax
from jax.experimental import pallas as pl
from jax.experimental.pallas import tpu as pltpu, tpu_sc as plsc

_D = 768
_BATCH = 4
_SEQ = 2048
_ROWS = _BATCH * _SEQ

_NW = 32
_PW = _SEQ // _NW
_VPR = _D // 16


def _sinusoidal_pe(max_len, d_model):
    pos = np.arange(max_len, dtype=np.float32)[:, None]
    div = np.exp(
        np.arange(0, d_model, 2, dtype=np.float32) * (-np.log(10000.0) / d_model)
    )
    pe = np.zeros((max_len, d_model), dtype=np.float32)
    pe[:, 0::2] = np.sin(pos * div)
    pe[:, 1::2] = np.cos(pos * div)
    return jnp.asarray(pe)


_PE = _sinusoidal_pe(_SEQ, _D)

_mesh = plsc.VectorSubcoreMesh(core_axis_name="c", subcore_axis_name="s")


@functools.partial(
    pl.kernel,
    out_type=jax.ShapeDtypeStruct((_ROWS, _D), jnp.float32),
    mesh=_mesh,
    scratch_types=[
        pltpu.VMEM((_BATCH, _PW), jnp.int32),
        pltpu.VMEM((_PW, _D), jnp.float32),
        pltpu.VMEM((_PW, _D), jnp.float32),
        pltpu.SemaphoreType.DMA,
    ],
)
def _emb_kernel(idx_hbm, table_hbm, pe_hbm, out_hbm, idx_v, pe_buf, rows, sem):
    wid = lax.axis_index("s") * 2 + lax.axis_index("c")
    pbase = wid * _PW
    for b in range(_BATCH):
        pltpu.sync_copy(idx_hbm.at[b * _NW + wid], idx_v.at[b])
    pltpu.sync_copy(pe_hbm.at[pl.ds(pbase, _PW)], pe_buf)
    for b in range(_BATCH):
        pltpu.async_copy(table_hbm.at[idx_v.at[b]], rows, sem).wait()

        def add_row(r):
            for j in range(_VPR):
                sl = pl.ds(j * 16, 16)
                rows[r, sl] = rows[r, sl] + pe_buf[r, sl]

        lax.fori_loop(0, _PW, lambda r, _: (add_row(r), 0)[1], 0)
        pltpu.sync_copy(rows, out_hbm.at[pl.ds(b * _SEQ + pbase, _PW)])


def kernel(x, table):
    idx = x.reshape(_BATCH * _NW, _PW).astype(jnp.int32)
    out = _emb_kernel(idx, table, _PE)
    return out.reshape(_BATCH, _SEQ, _D)
